# baseline (speedup 1.0000x reference)
import functools

import jax
import jax.numpy as jnp
from jax.experimental import pallas as pl
from jax.experimental.pallas import tpu as pltpu

EPS = 1e-5
VMEM_LIMIT_BYTES = 48 * 1024 * 1024
B = 64


def _taps_ref3(src_e, src_o, patch_ref, Wp, Mext, C, delta):
    for j in range(9):
        kh, kw = divmod(j, 3)
        s = B + (kh - 1) * Wp + (kw - 1) + delta
        if s % 2 == 0:
            patch_ref[:, j * C:(j + 1) * C] = src_e[0, s:s + Mext, :]
        else:
            patch_ref[:, j * C:(j + 1) * C] = src_o[0, s + 1:s + 1 + Mext, :]


def _taps_ref2(src_e, src_o, patch_ref, Wp, Mext, C, delta):
    for j in range(9):
        kh, kw = divmod(j, 3)
        s = B + (kh - 1) * Wp + (kw - 1) + delta
        if s % 2 == 0:
            patch_ref[:, j * C:(j + 1) * C] = src_e[s:s + Mext, :]
        else:
            patch_ref[:, j * C:(j + 1) * C] = src_o[s + 1:s + 1 + Mext, :]


def _masked_stats(y, stat_ref, Wp, W, Mext):
    lane = jax.lax.broadcasted_iota(jnp.int32, (1, Mext), 1)
    mvec = (jax.lax.rem(lane, Wp) < W).astype(jnp.float32)
    s1 = jnp.dot(mvec, y, preferred_element_type=jnp.float32)
    s2 = jnp.dot(mvec, y * y, preferred_element_type=jnp.float32)
    C = y.shape[1]
    stat_ref[0:1, 0:1, :] = s1.reshape(1, 1, C)
    stat_ref[0:1, 1:2, :] = s2.reshape(1, 1, C)


def _conv1_kernel(Wp, W, Mext, xe_ref, xo_ref, w_ref, y_ref, stat_ref,
                  patch_ref):
    C = w_ref.shape[1]
    _taps_ref3(xe_ref, xo_ref, patch_ref, Wp, Mext, C, delta=1)
    y = jnp.dot(patch_ref[...], w_ref[...],
                preferred_element_type=jnp.float32)
    _masked_stats(y, stat_ref, Wp, W, Mext)
    y_ref[...] = y.astype(jnp.bfloat16).reshape(1, Mext, C)


def _conv2_kernel(Wp, W, Mext, y1_ref, s_ref, t_ref, w_ref, cm_ref,
                  y_ref, stat_ref, ae_ref, ao_ref, patch_ref):
    C = w_ref.shape[1]
    Mtot = ae_ref.shape[0]
    a = y1_ref[0].astype(jnp.float32) * s_ref[...] + t_ref[...]
    a = (jnp.maximum(a, 0.0).astype(jnp.bfloat16)) * cm_ref[...]
    zero = jnp.zeros((B, C), jnp.bfloat16)
    ae_ref[0:B, :] = zero
    ae_ref[B + Mext:Mtot, :] = zero
    ae_ref[B:B + Mext, :] = a
    ao_ref[0:B + 8, :] = jnp.zeros((B + 8, C), jnp.bfloat16)
    ao_ref[Mtot - B - 8:Mtot, :] = jnp.zeros((B + 8, C), jnp.bfloat16)
    ao_ref[B + 1:B + 1 + Mext, :] = a
    _taps_ref2(ae_ref, ao_ref, patch_ref, Wp, Mext, C, delta=0)
    y = jnp.dot(patch_ref[...], w_ref[...],
                preferred_element_type=jnp.float32)
    _masked_stats(y, stat_ref, Wp, W, Mext)
    y_ref[...] = y.astype(jnp.bfloat16).reshape(1, Mext, C)


def _epilogue_kernel(Mext, y2_ref, xo_ref, s_ref, t_ref, o_ref):
    C = y2_ref.shape[2]
    xres = xo_ref[0, B + 2:B + 2 + Mext, :].astype(jnp.float32)
    y = y2_ref[0].astype(jnp.float32) * s_ref[...] + t_ref[...] + xres
    o_ref[...] = jnp.maximum(y, 0.0).reshape(1, Mext, C)


def _finalize_bn(stat_partials, gamma, beta, count):
    s = jnp.sum(stat_partials, axis=0)
    mean = s[0] / count
    var = jnp.maximum(s[1] / count - mean * mean, 0.0)
    inv = jax.lax.rsqrt(var + EPS)
    scale = gamma * inv
    shift = beta - mean * scale
    C = scale.shape[0]
    return scale.reshape(1, C), shift.reshape(1, C)


@jax.jit
def _basic_block(x_nchw, w1, g1, b1, w2, g2, b2):
    N, Cin, H, W = x_nchw.shape
    C = w1.shape[-1]
    Wp = W + 2
    Mext = H * Wp
    Mtot = B + Mext + B

    xb = jnp.transpose(x_nchw, (0, 2, 3, 1)).astype(jnp.bfloat16)
    xf = jnp.pad(xb, ((0, 0), (0, 0), (1, 1), (0, 0))).reshape(N, Mext, C)
    xe = jnp.pad(xf, ((0, 0), (B, B), (0, 0)))
    xo = jnp.pad(xf, ((0, 0), (B + 1, B - 1), (0, 0)))

    w1m = w1.reshape(9 * Cin, C).astype(jnp.bfloat16)
    w2m = w2.reshape(9 * C, C).astype(jnp.bfloat16)
    colmask = jnp.broadcast_to(
        (jnp.arange(Wp) < W)[None, :, None], (H, Wp, C)
    ).reshape(Mext, C).astype(jnp.bfloat16)
    count = float(N * H * W)

    cparams = pltpu.CompilerParams(
        dimension_semantics=("parallel",),
        vmem_limit_bytes=VMEM_LIMIT_BYTES)

    flat_spec = pl.BlockSpec((1, Mtot, C), lambda n: (n, 0, 0))
    act_spec = pl.BlockSpec((1, Mext, C), lambda n: (n, 0, 0))
    stat_spec = pl.BlockSpec((1, 2, C), lambda n: (n, 0, 0))

    def resident_spec(shape):
        return pl.BlockSpec(shape, lambda n: (0,) * len(shape))

    y1, stat1 = pl.pallas_call(
        functools.partial(_conv1_kernel, Wp, W, Mext),
        grid=(N,),
        in_specs=[flat_spec, flat_spec, resident_spec((9 * Cin, C))],
        out_specs=(act_spec, stat_spec),
        out_shape=(jax.ShapeDtypeStruct((N, Mext, C), jnp.bfloat16),
                   jax.ShapeDtypeStruct((N, 2, C), jnp.float32)),
        scratch_shapes=[pltpu.VMEM((Mext, 9 * Cin), jnp.bfloat16)],
        compiler_params=cparams,
    )(xe, xo, w1m)

    scale1, shift1 = _finalize_bn(stat1, g1, b1, count)

    y2, stat2 = pl.pallas_call(
        functools.partial(_conv2_kernel, Wp, W, Mext),
        grid=(N,),
        in_specs=[act_spec, resident_spec((1, C)), resident_spec((1, C)),
                  resident_spec((9 * C, C)), resident_spec((Mext, C))],
        out_specs=(act_spec, stat_spec),
        out_shape=(jax.ShapeDtypeStruct((N, Mext, C), jnp.bfloat16),
                   jax.ShapeDtypeStruct((N, 2, C), jnp.float32)),
        scratch_shapes=[pltpu.VMEM((Mtot, C), jnp.bfloat16),
                        pltpu.VMEM((Mtot, C), jnp.bfloat16),
                        pltpu.VMEM((Mext, 9 * C), jnp.bfloat16)],
        compiler_params=cparams,
    )(y1, scale1, shift1, w2m, colmask)

    scale2, shift2 = _finalize_bn(stat2, g2, b2, count)

    out_ext = pl.pallas_call(
        functools.partial(_epilogue_kernel, Mext),
        grid=(N,),
        in_specs=[act_spec, flat_spec,
                  resident_spec((1, C)), resident_spec((1, C))],
        out_specs=act_spec,
        out_shape=jax.ShapeDtypeStruct((N, Mext, C), jnp.float32),
        compiler_params=cparams,
    )(y2, xo, scale2, shift2)

    out = out_ext.reshape(N, H, Wp, C)[:, :, :W, :]
    return jnp.transpose(out, (0, 3, 1, 2))


def kernel(x_nchw, w1, g1, b1, w2, g2, b2):
    return _basic_block(x_nchw, w1, g1, b1, w2, g2, b2)


# speedup vs baseline: 2.2142x; 2.2142x over previous
import jax
import jax.numpy as jnp
from jax.experimental import pallas as pl
from jax.experimental.pallas import tpu as pltpu

EPS = 1e-5
VMEM_LIMIT_BYTES = 48 * 1024 * 1024


def _zero_halo_and_fill(pad_ref, interior, H, W, C):
    Hp, Wp = H + 2, W + 2
    pad_ref[0:1, :, :] = jnp.zeros((1, Wp, C), jnp.float32)
    pad_ref[H + 1:H + 2, :, :] = jnp.zeros((1, Wp, C), jnp.float32)
    pad_ref[:, 0:1, :] = jnp.zeros((Hp, 1, C), jnp.float32)
    pad_ref[:, W + 1:W + 2, :] = jnp.zeros((Hp, 1, C), jnp.float32)
    pad_ref[1:H + 1, 1:W + 1, :] = interior


def _im2col_conv(pad_ref, patch_ref, w_ref, H, W, Cin):
    apad = pad_ref[...]
    for j in range(9):
        kh, kw = divmod(j, 3)
        patch_ref[:, j * Cin:(j + 1) * Cin] = (
            apad[kh:kh + H, kw:kw + W, :].reshape(H * W, Cin))
    return jnp.dot(patch_ref[...], w_ref[...],
                   preferred_element_type=jnp.float32)


def _write_stats(stat_ref, y, Cout):
    stat_ref[0:1, 0:1, :] = jnp.sum(y, axis=0, keepdims=True).reshape(1, 1, Cout)
    stat_ref[0:1, 1:2, :] = jnp.sum(y * y, axis=0, keepdims=True).reshape(1, 1, Cout)


def _conv1_kernel(x_ref, w1_ref, y1_ref, stat1_ref, xpad_ref, patch_ref):
    _, H, W, Cin = x_ref.shape
    Cout = w1_ref.shape[1]
    xf = x_ref[...].reshape(H, W, Cin).astype(jnp.float32)
    _zero_halo_and_fill(xpad_ref, xf, H, W, Cin)
    y = _im2col_conv(xpad_ref, patch_ref, w1_ref, H, W, Cin)
    _write_stats(stat1_ref, y, Cout)
    y1_ref[...] = y.astype(jnp.bfloat16).reshape(1, H, W, Cout)


def _conv2_kernel(y1_ref, scale1_ref, shift1_ref, w2_ref,
                  y2_ref, stat2_ref, apad_ref, patch_ref):
    _, H, W, C = y1_ref.shape
    a = (y1_ref[...].reshape(H, W, C).astype(jnp.float32) * scale1_ref[...]
         + shift1_ref[...])
    a = jnp.maximum(a, 0.0)
    _zero_halo_and_fill(apad_ref, a, H, W, C)
    y = _im2col_conv(apad_ref, patch_ref, w2_ref, H, W, C)
    _write_stats(stat2_ref, y, C)
    y2_ref[...] = y.astype(jnp.bfloat16).reshape(1, H, W, C)


def _epilogue_kernel(y2_ref, x_ref, scale2_ref, shift2_ref, o_ref):
    y = (y2_ref[...].astype(jnp.float32) * scale2_ref[...] + shift2_ref[...]
         + x_ref[...].astype(jnp.float32))
    o_ref[...] = jnp.maximum(y, 0.0).astype(jnp.bfloat16)


def _finalize_bn(stat_partials, gamma, beta, count):
    s = jnp.sum(stat_partials, axis=0)
    mean = s[0] / count
    var = jnp.maximum(s[1] / count - mean * mean, 0.0)
    inv = jax.lax.rsqrt(var + EPS)
    scale = gamma * inv
    shift = beta - mean * scale
    C = scale.shape[0]
    return scale.reshape(1, C), shift.reshape(1, C)


@jax.jit
def _basic_block(x_nchw, w1, g1, b1, w2, g2, b2):
    N, Cin, H, W = x_nchw.shape
    Cout = w1.shape[-1]

    x = jnp.transpose(x_nchw, (0, 2, 3, 1)).astype(jnp.bfloat16)
    w1m = w1.reshape(9 * Cin, Cout).astype(jnp.float32)
    w2m = w2.reshape(9 * Cout, Cout).astype(jnp.float32)
    count = float(N * H * W)

    cparams = pltpu.CompilerParams(
        dimension_semantics=("parallel",),
        vmem_limit_bytes=VMEM_LIMIT_BYTES)

    def act_spec(C):
        return pl.BlockSpec((1, H, W, C), lambda n: (n, 0, 0, 0))

    def resident_spec(shape):
        return pl.BlockSpec(shape, lambda n: (0,) * len(shape))

    stat_spec = pl.BlockSpec((1, 2, Cout), lambda n: (n, 0, 0))

    y1, stat1 = pl.pallas_call(
        _conv1_kernel,
        grid=(N,),
        in_specs=[act_spec(Cin), resident_spec((9 * Cin, Cout))],
        out_specs=(act_spec(Cout), stat_spec),
        out_shape=(jax.ShapeDtypeStruct((N, H, W, Cout), jnp.bfloat16),
                   jax.ShapeDtypeStruct((N, 2, Cout), jnp.float32)),
        scratch_shapes=[pltpu.VMEM((H + 2, W + 2, Cin), jnp.float32),
                        pltpu.VMEM((H * W, 9 * Cin), jnp.float32)],
        compiler_params=cparams,
    )(x, w1m)

    scale1, shift1 = _finalize_bn(stat1, g1, b1, count)

    y2, stat2 = pl.pallas_call(
        _conv2_kernel,
        grid=(N,),
        in_specs=[act_spec(Cout), resident_spec((1, Cout)),
                  resident_spec((1, Cout)), resident_spec((9 * Cout, Cout))],
        out_specs=(act_spec(Cout), stat_spec),
        out_shape=(jax.ShapeDtypeStruct((N, H, W, Cout), jnp.bfloat16),
                   jax.ShapeDtypeStruct((N, 2, Cout), jnp.float32)),
        scratch_shapes=[pltpu.VMEM((H + 2, W + 2, Cout), jnp.float32),
                        pltpu.VMEM((H * W, 9 * Cout), jnp.float32)],
        compiler_params=cparams,
    )(y1, scale1, shift1, w2m)

    scale2, shift2 = _finalize_bn(stat2, g2, b2, count)

    out_nhwc = pl.pallas_call(
        _epilogue_kernel,
        grid=(N,),
        in_specs=[act_spec(Cout), act_spec(Cin),
                  resident_spec((1, Cout)), resident_spec((1, Cout))],
        out_specs=act_spec(Cout),
        out_shape=jax.ShapeDtypeStruct((N, H, W, Cout), jnp.bfloat16),
        compiler_params=cparams,
    )(y2, x, scale2, shift2)

    return jnp.transpose(out_nhwc, (0, 3, 1, 2)).astype(jnp.float32)


def kernel(x_nchw, w1, g1, b1, w2, g2, b2):
    return _basic_block(x_nchw, w1, g1, b1, w2, g2, b2)


# speedup vs baseline: 2.6869x; 1.2135x over previous
import jax
import jax.numpy as jnp
from jax.experimental import pallas as pl
from jax.experimental.pallas import tpu as pltpu

EPS = 1e-5
VMEM_LIMIT_BYTES = 48 * 1024 * 1024


def _zero_halo_and_fill(pad_ref, interior, H, W, C):
    Hp, Wp = H + 2, W + 2
    pad_ref[0:1, :, :] = jnp.zeros((1, Wp, C), jnp.float32)
    pad_ref[H + 1:H + 2, :, :] = jnp.zeros((1, Wp, C), jnp.float32)
    pad_ref[:, 0:1, :] = jnp.zeros((Hp, 1, C), jnp.float32)
    pad_ref[:, W + 1:W + 2, :] = jnp.zeros((Hp, 1, C), jnp.float32)
    pad_ref[1:H + 1, 1:W + 1, :] = interior


def _im2col_conv(pad_ref, patch_ref, w_ref, H, W, Cin):
    apad = pad_ref[...]
    for j in range(9):
        kh, kw = divmod(j, 3)
        patch_ref[:, j * Cin:(j + 1) * Cin] = (
            apad[kh:kh + H, kw:kw + W, :].reshape(H * W, Cin))
    return jnp.dot(patch_ref[...], w_ref[...],
                   preferred_element_type=jnp.float32)


def _write_stats(stat_ref, y, Cout):
    stat_ref[0:1, 0:1, :] = jnp.sum(y, axis=0, keepdims=True).reshape(1, 1, Cout)
    stat_ref[0:1, 1:2, :] = jnp.sum(y * y, axis=0, keepdims=True).reshape(1, 1, Cout)


def _conv1_kernel(x_ref, w1_ref, y1_ref, stat1_ref, xpad_ref, patch_ref):
    _, H, W, Cin = x_ref.shape
    Cout = w1_ref.shape[1]
    xf = x_ref[...].reshape(H, W, Cin)
    _zero_halo_and_fill(xpad_ref, xf, H, W, Cin)
    y = _im2col_conv(xpad_ref, patch_ref, w1_ref, H, W, Cin)
    _write_stats(stat1_ref, y, Cout)
    y1_ref[...] = y.astype(jnp.bfloat16).reshape(1, H, W, Cout)


def _conv2_kernel(y1_ref, scale1_ref, shift1_ref, w2_ref,
                  y2_ref, stat2_ref, apad_ref, patch_ref):
    _, H, W, C = y1_ref.shape
    a = (y1_ref[...].reshape(H, W, C).astype(jnp.float32) * scale1_ref[...]
         + shift1_ref[...])
    a = jnp.maximum(a, 0.0)
    _zero_halo_and_fill(apad_ref, a, H, W, C)
    y = _im2col_conv(apad_ref, patch_ref, w2_ref, H, W, C)
    _write_stats(stat2_ref, y, C)
    y2_ref[...] = y.astype(jnp.bfloat16).reshape(1, H, W, C)


def _epilogue_kernel(y2_ref, x_ref, scale2_ref, shift2_ref, o_ref):
    y = (y2_ref[...].astype(jnp.float32) * scale2_ref[...] + shift2_ref[...]
         + x_ref[...])
    o_ref[...] = jnp.maximum(y, 0.0)


def _finalize_bn(stat_partials, gamma, beta, count):
    s = jnp.sum(stat_partials, axis=0)
    mean = s[0] / count
    var = jnp.maximum(s[1] / count - mean * mean, 0.0)
    inv = jax.lax.rsqrt(var + EPS)
    scale = gamma * inv
    shift = beta - mean * scale
    C = scale.shape[0]
    return scale.reshape(1, C), shift.reshape(1, C)


@jax.jit
def _basic_block(x_nchw, w1, g1, b1, w2, g2, b2):
    N, Cin, H, W = x_nchw.shape
    Cout = w1.shape[-1]

    x = jnp.transpose(x_nchw, (0, 2, 3, 1)).astype(jnp.float32)
    w1m = w1.reshape(9 * Cin, Cout).astype(jnp.float32)
    w2m = w2.reshape(9 * Cout, Cout).astype(jnp.float32)
    count = float(N * H * W)

    cparams = pltpu.CompilerParams(
        dimension_semantics=("parallel",),
        vmem_limit_bytes=VMEM_LIMIT_BYTES)

    def act_spec(C):
        return pl.BlockSpec((1, H, W, C), lambda n: (n, 0, 0, 0))

    def resident_spec(shape):
        return pl.BlockSpec(shape, lambda n: (0,) * len(shape))

    stat_spec = pl.BlockSpec((1, 2, Cout), lambda n: (n, 0, 0))

    y1, stat1 = pl.pallas_call(
        _conv1_kernel,
        grid=(N,),
        in_specs=[act_spec(Cin), resident_spec((9 * Cin, Cout))],
        out_specs=(act_spec(Cout), stat_spec),
        out_shape=(jax.ShapeDtypeStruct((N, H, W, Cout), jnp.bfloat16),
                   jax.ShapeDtypeStruct((N, 2, Cout), jnp.float32)),
        scratch_shapes=[pltpu.VMEM((H + 2, W + 2, Cin), jnp.float32),
                        pltpu.VMEM((H * W, 9 * Cin), jnp.float32)],
        compiler_params=cparams,
    )(x, w1m)

    scale1, shift1 = _finalize_bn(stat1, g1, b1, count)

    y2, stat2 = pl.pallas_call(
        _conv2_kernel,
        grid=(N,),
        in_specs=[act_spec(Cout), resident_spec((1, Cout)),
                  resident_spec((1, Cout)), resident_spec((9 * Cout, Cout))],
        out_specs=(act_spec(Cout), stat_spec),
        out_shape=(jax.ShapeDtypeStruct((N, H, W, Cout), jnp.bfloat16),
                   jax.ShapeDtypeStruct((N, 2, Cout), jnp.float32)),
        scratch_shapes=[pltpu.VMEM((H + 2, W + 2, Cout), jnp.float32),
                        pltpu.VMEM((H * W, 9 * Cout), jnp.float32)],
        compiler_params=cparams,
    )(y1, scale1, shift1, w2m)

    scale2, shift2 = _finalize_bn(stat2, g2, b2, count)

    out_nhwc = pl.pallas_call(
        _epilogue_kernel,
        grid=(N,),
        in_specs=[act_spec(Cout), act_spec(Cin),
                  resident_spec((1, Cout)), resident_spec((1, Cout))],
        out_specs=act_spec(Cout),
        out_shape=jax.ShapeDtypeStruct((N, H, W, Cout), jnp.float32),
        compiler_params=cparams,
    )(y2, x, scale2, shift2)

    return jnp.transpose(out_nhwc, (0, 3, 1, 2))


def kernel(x_nchw, w1, g1, b1, w2, g2, b2):
    return _basic_block(x_nchw, w1, g1, b1, w2, g2, b2)
